# baseline (speedup 1.0000x reference)
import jax
import jax.numpy as jnp
from jax.experimental import pallas as pl
from jax.experimental.pallas import tpu as pltpu

_BLOCK = 40000


def _fused_mlp(x_ref, w1_ref, b1_ref, w2_ref, b2_ref, o_ref):
    x = x_ref[...]
    hT = jax.lax.dot_general(
        w1_ref[...], x,
        dimension_numbers=(((1,), (1,)), ((), ())),
        preferred_element_type=jnp.float32,
    )
    hT = jnp.maximum(hT + b1_ref[...], 0.0)
    y = jax.lax.dot_general(
        w2_ref[...], hT,
        dimension_numbers=(((1,), (0,)), ((), ())),
        preferred_element_type=jnp.float32,
    )
    o_ref[0, :, :] = y + b2_ref[0, 0]


def kernel(embeddings, W1, b1, W2, b2):
    n, d = embeddings.shape
    hdim = W1.shape[0]
    b1r = b1.reshape(hdim, 1)
    b2r = b2.reshape(1, 1)
    nb = n // _BLOCK
    out = pl.pallas_call(
        _fused_mlp,
        grid=(nb,),
        in_specs=[
            pl.BlockSpec((_BLOCK, d), lambda i: (i, 0)),
            pl.BlockSpec((hdim, d), lambda i: (0, 0)),
            pl.BlockSpec((hdim, 1), lambda i: (0, 0)),
            pl.BlockSpec((1, hdim), lambda i: (0, 0)),
            pl.BlockSpec((1, 1), lambda i: (0, 0)),
        ],
        out_specs=pl.BlockSpec((1, 1, _BLOCK), lambda i: (i, 0, 0)),
        out_shape=jax.ShapeDtypeStruct((nb, 1, _BLOCK), jnp.float32),
        compiler_params=pltpu.CompilerParams(
            dimension_semantics=("parallel",),
        ),
    )(embeddings, W1, b1r, W2, b2r)
    return out.reshape(n)
